# speedup vs baseline: 8.1327x; 8.1327x over previous
import functools

import jax
import jax.numpy as jnp
from jax import lax
from jax.experimental import pallas as pl
from jax.experimental.pallas import tpu as pltpu
from jax.experimental.pallas import tpu_sc as plsc

B = 16
T = 16384
D = 128
NR_PAD = 8
IDX_CHUNK = 128


def _sc_gather(flat_users, nodes_v, embed_u_w, embed_i_w):
    info = plsc.get_sparse_core_info()
    nc, ns = info.num_cores, info.num_subcores
    nw = nc * ns
    rows_per_w = T // nw
    n_chunks = rows_per_w // IDX_CHUNK

    mesh = plsc.VectorSubcoreMesh(core_axis_name="c", subcore_axis_name="s")

    @functools.partial(
        pl.kernel,
        mesh=mesh,
        out_type=[
            jax.ShapeDtypeStruct((T, D), jnp.float32),
            jax.ShapeDtypeStruct((B, D), jnp.float32),
        ],
        scratch_types=[
            pltpu.VMEM((rows_per_w,), jnp.int32),
            pltpu.VMEM((rows_per_w, D), jnp.float32),
            pltpu.VMEM((B,), jnp.int32),
            pltpu.VMEM((B, D), jnp.float32),
            pltpu.SemaphoreType.DMA,
            pltpu.SemaphoreType.DMA,
        ],
    )
    def gather_kernel(users_hbm, nodes_hbm, tab_u, tab_i, out_pt, out_qj,
                      idx_v, rows_v, nidx_v, qrows_v, sem, qsem):
        wid = lax.axis_index("s") * nc + lax.axis_index("c")
        base = wid * rows_per_w
        pltpu.sync_copy(users_hbm.at[pl.ds(base, rows_per_w)], idx_v)
        copies = []
        for c in range(n_chunks):
            copies.append(pltpu.async_copy(
                tab_u.at[idx_v.at[pl.ds(c * IDX_CHUNK, IDX_CHUNK)]],
                rows_v.at[pl.ds(c * IDX_CHUNK, IDX_CHUNK)], sem))
        for cp in copies:
            cp.wait()
        pltpu.sync_copy(rows_v, out_pt.at[pl.ds(base, rows_per_w)])

        @pl.when(wid == 0)
        def _():
            pltpu.sync_copy(nodes_hbm, nidx_v)
            pltpu.async_copy(tab_i.at[nidx_v], qrows_v, qsem).wait()
            pltpu.sync_copy(qrows_v, out_qj)

    return gather_kernel(flat_users, nodes_v, embed_u_w, embed_i_w)


_NT = (((1,), (1,)), ((), ()))
_TN = (((0,), (0,)), ((), ()))


def _tc_body(rat_ref, seg_ref, pt_ref, qj_ref, er_ref, g1_ref, g1b_ref,
             g2_ref, g2b_ref, a1_ref, a1b_ref, a2_ref, a2b_ref, a3_ref,
             a3b_ref, out_ref, m_ref, d_ref, z_ref, rtab_ref, stab_ref,
             *, tblk, nblk):
    i = pl.program_id(0)

    @pl.when(i == 0)
    def _init():
        m_ref[...] = jnp.full((B, 1), -1e30, jnp.float32)
        d_ref[...] = jnp.zeros((B, 1), jnp.float32)
        z_ref[...] = jnp.zeros((B, D), jnp.float32)
        rtab_ref[...] = lax.dot_general(er_ref[...], g1_ref[:, D:], _NT)
        stab_ref[...] = lax.dot_general(qj_ref[...], a1_ref[:, D:], _NT)

    pt = pt_ref[...]
    rat_row = rat_ref[...].reshape(1, tblk)
    seg_row = seg_ref[...].reshape(1, tblk)

    rat_ohT = (lax.broadcasted_iota(jnp.int32, (NR_PAD, tblk), 0)
               == rat_row).astype(jnp.float32)
    seg_ohT = lax.broadcasted_iota(jnp.int32, (B, tblk), 0) == seg_row

    er_c = lax.dot_general(rat_ohT, rtab_ref[...], _TN)
    h = jnp.maximum(lax.dot_general(pt, g1_ref[:, :D], _NT) + er_c
                    + g1b_ref[...], 0.0)
    fjt = jnp.maximum(lax.dot_general(h, g2_ref[...], _NT) + g2b_ref[...], 0.0)
    seg_c = lax.dot_general(seg_ohT.astype(jnp.float32), stab_ref[...], _TN)
    a = jnp.maximum(lax.dot_general(fjt, a1_ref[:, :D], _NT) + seg_c
                    + a1b_ref[...], 0.0)
    a = jnp.maximum(lax.dot_general(a, a2_ref[...], _NT) + a2b_ref[...], 0.0)
    s_row = lax.dot_general(a3_ref[...], a, _NT) + a3b_ref[...]

    neg = jnp.float32(-1e30)
    masked = jnp.where(seg_ohT, s_row, neg)
    blk_m = jnp.max(masked, axis=1, keepdims=True)
    m_old = m_ref[...]
    m_new = jnp.maximum(m_old, blk_m)
    scale = jnp.exp(m_old - m_new)
    e_t = jnp.exp(jnp.where(seg_ohT, s_row - m_new, neg))
    m_ref[...] = m_new
    d_ref[...] = d_ref[...] * scale + jnp.sum(e_t, axis=1, keepdims=True)
    z_ref[...] = (z_ref[...] * scale
                  + lax.dot_general(e_t, fjt, (((1,), (0,)), ((), ()))))

    @pl.when(i == nblk - 1)
    def _finish():
        dd = d_ref[...]
        out_ref[...] = jnp.where(dd > 0, z_ref[...] / dd, 0.0)


def _tc_fused(pt, qj, flat_ratings, segment_ids, embed_r_w,
              g1_w, g1_b, g2_w, g2_b, a1_w, a1_b, a2_w, a2_b, a3_w, a3_b):
    tblk = 2048
    nblk = T // tblk

    rat3 = flat_ratings.reshape(nblk, 1, tblk)
    seg3 = segment_ids.reshape(nblk, 1, tblk)
    er_pad = jnp.pad(embed_r_w, ((0, NR_PAD - embed_r_w.shape[0]), (0, 0)))

    full = lambda shape: pl.BlockSpec(shape, lambda i: tuple(0 for _ in shape))
    body = functools.partial(_tc_body, tblk=tblk, nblk=nblk)

    return pl.pallas_call(
        body,
        grid=(nblk,),
        in_specs=[
            pl.BlockSpec((1, 1, tblk), lambda i: (i, 0, 0)),
            pl.BlockSpec((1, 1, tblk), lambda i: (i, 0, 0)),
            pl.BlockSpec((tblk, D), lambda i: (i, 0)),
            full((B, D)),
            full((NR_PAD, D)),
            full((D, 2 * D)),
            full((1, D)),
            full((D, D)),
            full((1, D)),
            full((D, 2 * D)),
            full((1, D)),
            full((D, D)),
            full((1, D)),
            full((1, D)),
            full((1, 1)),
        ],
        out_specs=pl.BlockSpec((B, D), lambda i: (0, 0)),
        out_shape=jax.ShapeDtypeStruct((B, D), jnp.float32),
        scratch_shapes=[
            pltpu.VMEM((B, 1), jnp.float32),
            pltpu.VMEM((B, 1), jnp.float32),
            pltpu.VMEM((B, D), jnp.float32),
            pltpu.VMEM((NR_PAD, D), jnp.float32),
            pltpu.VMEM((B, D), jnp.float32),
        ],
    )(rat3, seg3, pt, qj, er_pad, g1_w, g1_b.reshape(1, D), g2_w,
      g2_b.reshape(1, D), a1_w, a1_b.reshape(1, D), a2_w, a2_b.reshape(1, D),
      a3_w, a3_b.reshape(1, 1))


def kernel(nodes_v, flat_users, flat_ratings, segment_ids, embed_u_w,
           embed_i_w, embed_r_w, g1_w, g1_b, g2_w, g2_b, a1_w, a1_b,
           a2_w, a2_b, a3_w, a3_b):
    pt, qj = _sc_gather(flat_users, nodes_v, embed_u_w, embed_i_w)
    return _tc_fused(pt, qj, flat_ratings, segment_ids, embed_r_w,
                     g1_w, g1_b, g2_w, g2_b, a1_w, a1_b, a2_w, a2_b,
                     a3_w, a3_b)
